# baseline (speedup 1.0000x reference)
import functools

import jax
import jax.numpy as jnp
from jax import lax
from jax.experimental import pallas as pl
from jax.experimental.pallas import tpu as pltpu
from jax.experimental.pallas import tpu_sc as plsc

_L = 16
_CHUNK = 4096


def _edge_loss_partials(vflat, bs, VC, v0p, v1p, num_cores, num_subcores):
    EP = v0p.shape[0]
    NW = num_cores * num_subcores
    bpw = bs // NW
    n_chunks = EP // _CHUNK

    mesh = plsc.VectorSubcoreMesh(core_axis_name="c", subcore_axis_name="s")

    @functools.partial(
        pl.kernel,
        mesh=mesh,
        compiler_params=pltpu.CompilerParams(
            needs_layout_passes=False, use_tc_tiling_on_sc=False
        ),
        out_type=jax.ShapeDtypeStruct((NW, _L), jnp.float32),
        scratch_types=[
            pltpu.VMEM((bpw * VC,), jnp.float32),
            pltpu.VMEM((_CHUNK,), jnp.int32),
            pltpu.VMEM((_CHUNK,), jnp.int32),
            pltpu.VMEM((_L,), jnp.float32),
        ],
    )
    def edge_loss_sc(v_hbm, v0_hbm, v1_hbm, out_hbm, vert_v, i0_v, i1_v, acc_v):
        wid = lax.axis_index("s") * num_cores + lax.axis_index("c")
        base = wid * bpw
        pltpu.sync_copy(v_hbm.at[pl.ds(base * VC, bpw * VC)], vert_v)

        acc = jnp.zeros((_L,), jnp.float32)
        for k in range(n_chunks):
            pltpu.sync_copy(v0_hbm.at[pl.ds(k * _CHUNK, _CHUNK)], i0_v)
            pltpu.sync_copy(v1_hbm.at[pl.ds(k * _CHUNK, _CHUNK)], i1_v)

            def body(j, acc):
                i0 = i0_v[pl.ds(j * _L, _L)] * 3
                i1 = i1_v[pl.ds(j * _L, _L)] * 3
                for b in range(bpw):
                    for c in range(3):
                        off = b * VC + c
                        a = plsc.load_gather(vert_v, [i0 + off])
                        bb = plsc.load_gather(vert_v, [i1 + off])
                        d = a - bb
                        acc = acc + d * d
                return acc

            acc = lax.fori_loop(0, _CHUNK // _L, body, acc)

        acc_v[...] = acc
        pltpu.sync_copy(acc_v, out_hbm.at[wid])

    return edge_loss_sc(vflat, v0p, v1p)


def kernel(vertices, v0, v1):
    bs, V, C = vertices.shape
    E = v0.shape[0]
    info = plsc.get_sparse_core_info()
    EP = ((E + _CHUNK - 1) // _CHUNK) * _CHUNK
    v0p = jnp.pad(v0.astype(jnp.int32), (0, EP - E))
    v1p = jnp.pad(v1.astype(jnp.int32), (0, EP - E))
    partials = _edge_loss_partials(
        vertices.reshape(-1), bs, V * C, v0p, v1p,
        info.num_cores, info.num_subcores,
    )
    return (partials.sum() / (E * bs)).astype(jnp.float32)


# speedup vs baseline: 23.9013x; 23.9013x over previous
import functools

import jax
import jax.numpy as jnp
from jax import lax
from jax.experimental import pallas as pl
from jax.experimental.pallas import tpu as pltpu
from jax.experimental.pallas import tpu_sc as plsc

_L = 16
_CHUNK = 4096


def _edge_loss_partials(vflat, bs, VC, v0p, v1p, num_cores, num_subcores):
    del bs
    bs = vflat.shape[0]
    EP = v0p.shape[0]
    NW = num_cores * num_subcores
    bpw = bs // NW
    n_chunks = EP // _CHUNK

    mesh = plsc.VectorSubcoreMesh(core_axis_name="c", subcore_axis_name="s")

    @functools.partial(
        pl.kernel,
        mesh=mesh,
        compiler_params=pltpu.CompilerParams(
            needs_layout_passes=False, use_tc_tiling_on_sc=False
        ),
        out_type=jax.ShapeDtypeStruct((NW, _L), jnp.float32),
        scratch_types=[
            pltpu.VMEM((bpw * VC,), jnp.float32),
            pltpu.VMEM((2, _CHUNK), jnp.int32),
            pltpu.VMEM((2, _CHUNK), jnp.int32),
            pltpu.VMEM((_L,), jnp.float32),
            pltpu.SemaphoreType.DMA,
            pltpu.SemaphoreType.DMA,
        ],
    )
    def edge_loss_sc(
        v_hbm, v0_hbm, v1_hbm, out_hbm, vert_v, i0_v, i1_v, acc_v, sem0, sem1
    ):
        wid = lax.axis_index("s") * num_cores + lax.axis_index("c")
        base = wid * bpw
        sems = (sem0, sem1)

        def issue(k):
            p = k % 2
            h0 = pltpu.async_copy(
                v0_hbm.at[pl.ds(k * _CHUNK, _CHUNK)], i0_v.at[p], sems[p]
            )
            h1 = pltpu.async_copy(
                v1_hbm.at[pl.ds(k * _CHUNK, _CHUNK)], i1_v.at[p], sems[p]
            )
            return h0, h1

        pending = {0: issue(0)}
        for b in range(bpw):
            pltpu.sync_copy(
                v_hbm.at[base + b], vert_v.at[pl.ds(b * VC, VC)]
            )

        n_acc = bpw * 3
        accs = [jnp.zeros((_L,), jnp.float32)] * n_acc
        for k in range(n_chunks):
            p = k % 2
            h0, h1 = pending.pop(k)
            h0.wait()
            h1.wait()
            if k + 1 < n_chunks:
                pending[k + 1] = issue(k + 1)

            def body(j, accs):
                accs = list(accs)
                i0 = i0_v[p, pl.ds(j * _L, _L)] * 3
                i1 = i1_v[p, pl.ds(j * _L, _L)] * 3
                for b in range(bpw):
                    for c in range(3):
                        off = b * VC + c
                        a = plsc.load_gather(vert_v, [i0 + off])
                        bb = plsc.load_gather(vert_v, [i1 + off])
                        d = a - bb
                        accs[b * 3 + c] = accs[b * 3 + c] + d * d
                return tuple(accs)

            accs = lax.fori_loop(
                0, _CHUNK // _L, body, tuple(accs), unroll=2
            )

        total = accs[0]
        for a in accs[1:]:
            total = total + a
        acc_v[...] = total
        pltpu.sync_copy(acc_v, out_hbm.at[wid])

    return edge_loss_sc(vflat, v0p, v1p)


def kernel(vertices, v0, v1):
    bs, V, C = vertices.shape
    E = v0.shape[0]
    info = plsc.get_sparse_core_info()
    EP = ((E + _CHUNK - 1) // _CHUNK) * _CHUNK
    v0p = jnp.pad(v0.astype(jnp.int32), (0, EP - E))
    v1p = jnp.pad(v1.astype(jnp.int32), (0, EP - E))
    partials = _edge_loss_partials(
        vertices.reshape(bs, V * C), bs, V * C, v0p, v1p,
        info.num_cores, info.num_subcores,
    )
    return (partials.sum() / (E * bs)).astype(jnp.float32)
